# Initial kernel scaffold; baseline (speedup 1.0000x reference)
import jax
import jax.numpy as jnp
from jax.experimental import pallas as pl


def kernel(logits, y_true):
    raise NotImplementedError("write your pallas kernel here")


# speedup vs baseline: 43.8040x; 43.8040x over previous
import functools

import jax
import jax.numpy as jnp
from jax import lax
from jax.experimental import pallas as pl
from jax.experimental.pallas import tpu as pltpu
from jax.experimental.pallas import tpu_sc as plsc

B = 128
V = 100000
NW = 32
RPW = B // NW
LANES = 16
NVREG = V // LANES
NEG = float("-inf")


def _merge_sorted3(a, b, c, a2, b2, c2):
    x1 = jnp.maximum(a, a2)
    y1 = jnp.minimum(a, a2)
    x2 = jnp.maximum(b, b2)
    y2 = jnp.minimum(b, b2)
    x3 = jnp.maximum(c, c2)
    m2 = jnp.maximum(y1, x2)
    m3 = jnp.maximum(jnp.minimum(y1, x2), jnp.maximum(y2, x3))
    return x1, m2, m3


def _make_sc_call():
    mesh = plsc.VectorSubcoreMesh(core_axis_name="c", subcore_axis_name="s")

    @functools.partial(
        pl.kernel,
        mesh=mesh,
        compiler_params=pltpu.CompilerParams(needs_layout_passes=False),
        out_type=jax.ShapeDtypeStruct((NW, LANES), jnp.float32),
        scratch_types=[
            pltpu.VMEM((V,), jnp.float32),
            pltpu.VMEM((LANES,), jnp.int32),
            pltpu.VMEM((LANES,), jnp.float32),
            pltpu.VMEM((LANES,), jnp.float32),
            pltpu.VMEM((LANES,), jnp.float32),
            pltpu.VMEM((LANES,), jnp.float32),
        ],
    )
    def dlr_loss_sc(logits_hbm, ypad_hbm, out_hbm, buf, yv, av, bv, cv, outv):
        wid = lax.axis_index("s") * 2 + lax.axis_index("c")
        pltpu.sync_copy(ypad_hbm.at[wid], yv)
        yvec = yv[...]
        iota = lax.iota(jnp.int32, LANES)
        out_acc = jnp.zeros((LANES,), jnp.float32)
        for r in range(RPW):
            row = wid * RPW + r
            pltpu.sync_copy(logits_hbm.at[row], buf)

            def body(j, carry):
                a, b, c = carry
                x = buf[pl.ds(j * LANES, LANES)]
                a2 = jnp.maximum(a, x)
                t = jnp.minimum(a, x)
                b2 = jnp.maximum(b, t)
                t2 = jnp.minimum(b, t)
                c2 = jnp.maximum(c, t2)
                return (a2, b2, c2)

            ninf = jnp.full((LANES,), NEG, dtype=jnp.float32)
            a, b, c = lax.fori_loop(0, NVREG, body, (ninf, ninf, ninf))

            for off in (8, 4, 2, 1):
                av[...] = a
                bv[...] = b
                cv[...] = c
                ix = jnp.bitwise_xor(iota, off)
                a_s = plsc.load_gather(av, [ix])
                b_s = plsc.load_gather(bv, [ix])
                c_s = plsc.load_gather(cv, [ix])
                a, b, c = _merge_sorted3(a, b, c, a_s, b_s, c_s)

            zy = plsc.load_gather(buf, [yvec])
            z_other = jnp.where((zy == a) & (b < a), b, a)
            scale = a - c + jnp.float32(1e-12)
            loss_vec = -(zy - z_other) / scale
            out_acc = jnp.where(iota == r, loss_vec, out_acc)
        outv[...] = out_acc
        pltpu.sync_copy(outv, out_hbm.at[wid])

    return dlr_loss_sc


_sc_call = _make_sc_call()


def kernel(logits, y_true):
    y32 = y_true.astype(jnp.int32)
    ypad = jnp.zeros((NW, LANES), jnp.int32).at[:, :RPW].set(
        y32.reshape(NW, RPW))
    out = _sc_call(logits, ypad)
    return out[:, :RPW].reshape(B)


# speedup vs baseline: 78.1763x; 1.7847x over previous
import functools

import jax
import jax.numpy as jnp
from jax import lax
from jax.experimental import pallas as pl
from jax.experimental.pallas import tpu as pltpu
from jax.experimental.pallas import tpu_sc as plsc

B = 128
V = 100000
NW = 32
RPW = B // NW
LANES = 16
NVREG = V // LANES
NTRIO = 5
NEG = float("-inf")


def _merge_sorted3(a, b, c, a2, b2, c2):
    x1 = jnp.maximum(a, a2)
    y1 = jnp.minimum(a, a2)
    x2 = jnp.maximum(b, b2)
    y2 = jnp.minimum(b, b2)
    x3 = jnp.maximum(c, c2)
    m2 = jnp.maximum(y1, x2)
    m3 = jnp.maximum(jnp.minimum(y1, x2), jnp.maximum(y2, x3))
    return x1, m2, m3


def _make_sc_call():
    mesh = plsc.VectorSubcoreMesh(core_axis_name="c", subcore_axis_name="s")

    @functools.partial(
        pl.kernel,
        mesh=mesh,
        compiler_params=pltpu.CompilerParams(needs_layout_passes=False),
        out_type=jax.ShapeDtypeStruct((NW, LANES), jnp.float32),
        scratch_types=[
            pltpu.VMEM((V,), jnp.float32),
            pltpu.VMEM((LANES,), jnp.int32),
            pltpu.VMEM((LANES,), jnp.float32),
            pltpu.VMEM((LANES,), jnp.float32),
            pltpu.VMEM((LANES,), jnp.float32),
            pltpu.VMEM((LANES,), jnp.float32),
        ],
    )
    def dlr_loss_sc(logits_hbm, ypad_hbm, out_hbm, buf, yv, av, bv, cv, outv):
        wid = lax.axis_index("s") * 2 + lax.axis_index("c")
        pltpu.sync_copy(ypad_hbm.at[wid], yv)
        yvec = yv[...]
        iota = lax.iota(jnp.int32, LANES)
        out_acc = jnp.zeros((LANES,), jnp.float32)
        for r in range(RPW):
            row = wid * RPW + r
            pltpu.sync_copy(logits_hbm.at[row], buf)

            def body(j, carry):
                new = []
                base = j * (LANES * NTRIO)
                for t in range(NTRIO):
                    a, b, c = carry[3 * t:3 * t + 3]
                    x = buf[pl.ds(base + t * LANES, LANES)]
                    a2 = jnp.maximum(a, x)
                    tt = jnp.minimum(a, x)
                    b2 = jnp.maximum(b, tt)
                    tt2 = jnp.minimum(b, tt)
                    c2 = jnp.maximum(c, tt2)
                    new += [a2, b2, c2]
                return tuple(new)

            ninf = jnp.full((LANES,), NEG, dtype=jnp.float32)
            trios = lax.fori_loop(0, NVREG // NTRIO, body,
                                  (ninf,) * (3 * NTRIO))
            a, b, c = trios[0:3]
            for t in range(1, NTRIO):
                a, b, c = _merge_sorted3(a, b, c, *trios[3 * t:3 * t + 3])

            for off in (8, 4, 2, 1):
                av[...] = a
                bv[...] = b
                cv[...] = c
                ix = jnp.bitwise_xor(iota, off)
                a_s = plsc.load_gather(av, [ix])
                b_s = plsc.load_gather(bv, [ix])
                c_s = plsc.load_gather(cv, [ix])
                a, b, c = _merge_sorted3(a, b, c, a_s, b_s, c_s)

            zy = plsc.load_gather(buf, [yvec])
            z_other = jnp.where((zy == a) & (b < a), b, a)
            scale = a - c + jnp.float32(1e-12)
            loss_vec = -(zy - z_other) / scale
            out_acc = jnp.where(iota == r, loss_vec, out_acc)
        outv[...] = out_acc
        pltpu.sync_copy(outv, out_hbm.at[wid])

    return dlr_loss_sc


_sc_call = _make_sc_call()


def kernel(logits, y_true):
    y32 = y_true.astype(jnp.int32)
    ypad = jnp.zeros((NW, LANES), jnp.int32).at[:, :RPW].set(
        y32.reshape(NW, RPW))
    out = _sc_call(logits, ypad)
    return out[:, :RPW].reshape(B)
